# baseline (speedup 1.0000x reference)
import functools

import jax
import jax.numpy as jnp
from jax import lax
from jax.experimental import pallas as pl
from jax.experimental.pallas import tpu as pltpu
from jax.experimental.pallas import tpu_sc as plsc

V = 1000
VP = 1024
B = 32
T = 1024
NTOK = B * T

_info = plsc.get_sparse_core_info()
NC = _info.num_cores
NS = _info.num_subcores
L = _info.num_lanes
NW = NC * NS
BPW = NTOK // NW
CH = 8
NB = 8
NCH = BPW // CH
PK = 128


def _row_lse_body(tab_ref, out_ref):
    x = tab_ref[...]
    m = jnp.max(x, axis=1)
    s = jnp.sum(jnp.exp(x - m[:, None]), axis=1)
    out_ref[...] = jnp.log(s) + m


def _row_lse(table):
    return pl.pallas_call(
        _row_lse_body,
        out_shape=jax.ShapeDtypeStruct((V,), jnp.float32),
    )(table)


@functools.partial(
    pl.kernel,
    mesh=plsc.VectorSubcoreMesh(core_axis_name="c", subcore_axis_name="s"),
    compiler_params=pltpu.CompilerParams(
        use_tc_tiling_on_sc=True, needs_layout_passes=False),
    out_type=[
        jax.ShapeDtypeStruct((NTOK, VP), jnp.float32),
        jax.ShapeDtypeStruct((NW, L), jnp.float32),
    ],
    scratch_types=[
        pltpu.VMEM((BPW,), jnp.int32),
        pltpu.VMEM((BPW,), jnp.int32),
        pltpu.VMEM((BPW,), jnp.int32),
        pltpu.VMEM((NB, CH, VP), jnp.float32),
        pltpu.VMEM((BPW,), jnp.float32),
        pltpu.VMEM((V,), jnp.float32),
        pltpu.VMEM((L,), jnp.float32),
        pltpu.SemaphoreType.DMA((NB,)),
        pltpu.SemaphoreType.DMA((NB,)),
        pltpu.SemaphoreType.DMA,
    ],
)
def _sc_gather(idx_hbm, tgt_hbm, lse_hbm, table_hbm, tflat_hbm,
               out_hbm, part_hbm,
               idx_v, tgt_v, fidx_v, rows_v,
               pick_v, lse_v, acc_v, gsem_v, wsem_v, psem):
    bufs = [rows_v.at[s] for s in range(NB)]
    gsems = [gsem_v.at[s] for s in range(NB)]
    wsems = [wsem_v.at[s] for s in range(NB)]
    wid = lax.axis_index("s") * NC + lax.axis_index("c")
    wbase = wid * BPW
    pltpu.sync_copy(idx_hbm.at[wid], idx_v)

    def gather(c, s):
        pltpu.async_copy(table_hbm.at[idx_v.at[pl.ds(c * CH, CH)]],
                         bufs[s], gsems[s])

    def writeout(c, s):
        pltpu.async_copy(bufs[s], out_hbm.at[pl.ds(wbase + c * CH, CH)],
                         wsems[s])

    def gwait(s):
        pltpu.make_async_copy(table_hbm.at[pl.ds(0, CH)], bufs[s],
                              gsems[s]).wait()

    def wwait(s):
        pltpu.make_async_copy(bufs[s], out_hbm.at[pl.ds(0, CH)],
                              wsems[s]).wait()

    for s in range(NB):
        gather(s, s)

    pltpu.sync_copy(tgt_hbm.at[wid], tgt_v)
    pltpu.sync_copy(lse_hbm, lse_v)

    def fidx_body(g, carry):
        sl = pl.ds(g * L, L)
        fidx_v[sl] = idx_v[sl] * VP + tgt_v[sl]
        return carry

    lax.fori_loop(0, BPW // L, fidx_body, 0)

    pick_cps = [
        pltpu.async_copy(tflat_hbm.at[fidx_v.at[pl.ds(j * PK, PK)]],
                         pick_v.at[pl.ds(j * PK, PK)], psem)
        for j in range(BPW // PK)
    ]
    nrounds = NCH // NB

    def round_body(r, carry):
        for s in range(NB):
            c = r * NB + s
            gwait(s)
            writeout(c, s)

            @pl.when(r < nrounds - 1)
            def _():
                wwait(s)
                gather(c + NB, s)

        return carry

    lax.fori_loop(0, nrounds, round_body, 0)
    for s in range(NB):
        wwait(s)

    for cp in pick_cps:
        cp.wait()
    acc_v[...] = jnp.full((L,), 0.0, jnp.float32)

    def loss_body(g, carry):
        sl = pl.ds(g * L, L)
        lse16 = plsc.load_gather(lse_v, [idx_v[sl]])
        acc_v[...] = acc_v[...] + (lse16 - pick_v[sl])
        return carry

    lax.fori_loop(0, BPW // L, loss_body, 0)
    pltpu.sync_copy(acc_v, part_hbm.at[wid])


def _finish_body(p_ref, out_ref):
    out_ref[...] = (jnp.sum(p_ref[...]) / jnp.float32(NTOK)).reshape(1, 1)


def _finish(partials):
    out = pl.pallas_call(
        _finish_body,
        out_shape=jax.ShapeDtypeStruct((1, 1), jnp.float32),
    )(partials)
    return out[0, 0]


def kernel(idx, targets, table):
    idx_f = idx.astype(jnp.int32)
    tgt_f = targets.astype(jnp.int32)
    table = table.astype(jnp.float32)
    table_p = jnp.pad(table, ((0, 0), (0, VP - V)))
    row_lse = _row_lse(table)
    tflat = jnp.concatenate(
        [table_p.reshape(V * VP), jnp.zeros(8, jnp.float32)])
    out, parts = _sc_gather(idx_f, tgt_f, row_lse, table_p, tflat)
    loss = _finish(parts)
    logits = out.reshape(B, T, VP)[:, :, :V]
    return (logits, loss)
